# Initial kernel scaffold; baseline (speedup 1.0000x reference)
import jax
import jax.numpy as jnp
from jax.experimental import pallas as pl


def kernel(x, adj, W1, a1, W2, a2):
    raise NotImplementedError("write your pallas kernel here")


# speedup vs baseline: 1.6155x; 1.6155x over previous
import jax
import jax.numpy as jnp
from jax.experimental import pallas as pl

B, K, IN, H, OUT = 8, 256, 128, 64, 128
ALPHA = 0.2
NEG_BIG = -9000000000000000.0


def _gat_block(h, adj, W_ref, a_src_ref, a_dst_ref):
    Wh = jax.lax.dot_general(h, W_ref[...], (((1,), (0,)), ((), ())),
                             preferred_element_type=jnp.float32)
    s = jax.lax.dot_general(Wh, a_src_ref[...], (((1,), (1,)), ((), ())),
                            preferred_element_type=jnp.float32)
    d = jax.lax.dot_general(a_dst_ref[...], Wh, (((1,), (1,)), ((), ())),
                            preferred_element_type=jnp.float32)
    e = s + d
    e = jnp.where(e >= 0, e, ALPHA * e)
    att = jnp.where(adj > 0, e, NEG_BIG)
    m = jnp.max(att, axis=1, keepdims=True)
    p = jnp.exp(att - m)
    att = p / jnp.sum(p, axis=1, keepdims=True)
    hp = jax.lax.dot_general(att, Wh, (((1,), (0,)), ((), ())),
                             preferred_element_type=jnp.float32)
    return jnp.where(hp > 0, hp, jnp.exp(jnp.minimum(hp, 0.0)) - 1.0)


def _gat2_kernel(x_ref, adj_ref, W1_ref, a1s_ref, a1d_ref,
                 W2_ref, a2s_ref, a2d_ref, out_ref):
    x = x_ref[0]
    adj = adj_ref[0]
    h1 = _gat_block(x, adj, W1_ref, a1s_ref, a1d_ref)
    out_ref[0] = _gat_block(h1, adj, W2_ref, a2s_ref, a2d_ref)


def kernel(x, adj, W1, a1, W2, a2):
    a1s = a1[:H][None, :]
    a1d = a1[H:][None, :]
    a2s = a2[:OUT][None, :]
    a2d = a2[OUT:][None, :]
    grid = (B,)
    out = pl.pallas_call(
        _gat2_kernel,
        grid=grid,
        in_specs=[
            pl.BlockSpec((1, K, IN), lambda b: (b, 0, 0)),
            pl.BlockSpec((1, K, K), lambda b: (b, 0, 0)),
            pl.BlockSpec((IN, H), lambda b: (0, 0)),
            pl.BlockSpec((1, H), lambda b: (0, 0)),
            pl.BlockSpec((1, H), lambda b: (0, 0)),
            pl.BlockSpec((H, OUT), lambda b: (0, 0)),
            pl.BlockSpec((1, OUT), lambda b: (0, 0)),
            pl.BlockSpec((1, OUT), lambda b: (0, 0)),
        ],
        out_specs=pl.BlockSpec((1, K, OUT), lambda b: (b, 0, 0)),
        out_shape=jax.ShapeDtypeStruct((B, K, OUT), jnp.float32),
    )(x, adj, W1, a1s, a1d, W2, a2s, a2d)
    return out


# speedup vs baseline: 2.0296x; 1.2563x over previous
import jax
import jax.numpy as jnp
from jax.experimental import pallas as pl

B, K, IN, H, OUT = 8, 256, 128, 64, 128
BPG = 4
ALPHA = 0.2
NEG_BIG = -9000000000000000.0


def _gat_block(h, adj, W_ref, a_src_ref, a_dst_ref, ones_ref):
    Wh = jax.lax.dot_general(h, W_ref[...], (((1,), (0,)), ((), ())),
                             preferred_element_type=jnp.float32)
    s = jax.lax.dot_general(Wh, a_src_ref[...], (((1,), (1,)), ((), ())),
                            preferred_element_type=jnp.float32)
    d = jax.lax.dot_general(a_dst_ref[...], Wh, (((1,), (1,)), ((), ())),
                            preferred_element_type=jnp.float32)
    e = s + d
    e = jnp.maximum(e, ALPHA * e)
    att = jnp.where(adj > 0, e, NEG_BIG)
    m = jnp.max(att, axis=1, keepdims=True)
    p = jnp.exp(att - m)
    rs = jax.lax.dot_general(p, ones_ref[...], (((1,), (1,)), ((), ())),
                             preferred_element_type=jnp.float32)
    hp = jax.lax.dot_general(p, Wh, (((1,), (0,)), ((), ())),
                             preferred_element_type=jnp.float32)
    hp = hp * (1.0 / rs)
    return jnp.where(hp > 0, hp, jnp.exp(jnp.minimum(hp, 0.0)) - 1.0)


def _gat2_kernel(x_ref, adj_ref, W1_ref, a1s_ref, a1d_ref,
                 W2_ref, a2s_ref, a2d_ref, ones_ref, out_ref):
    for i in range(BPG):
        x = x_ref[i]
        adj = adj_ref[i]
        h1 = _gat_block(x, adj, W1_ref, a1s_ref, a1d_ref, ones_ref)
        out_ref[i] = _gat_block(h1, adj, W2_ref, a2s_ref, a2d_ref, ones_ref)


def kernel(x, adj, W1, a1, W2, a2):
    a1s = a1[:H][None, :]
    a1d = a1[H:][None, :]
    a2s = a2[:OUT][None, :]
    a2d = a2[OUT:][None, :]
    ones = jnp.ones((1, K), dtype=jnp.float32)
    grid = (B // BPG,)
    out = pl.pallas_call(
        _gat2_kernel,
        grid=grid,
        in_specs=[
            pl.BlockSpec((BPG, K, IN), lambda b: (b, 0, 0)),
            pl.BlockSpec((BPG, K, K), lambda b: (b, 0, 0)),
            pl.BlockSpec((IN, H), lambda b: (0, 0)),
            pl.BlockSpec((1, H), lambda b: (0, 0)),
            pl.BlockSpec((1, H), lambda b: (0, 0)),
            pl.BlockSpec((H, OUT), lambda b: (0, 0)),
            pl.BlockSpec((1, OUT), lambda b: (0, 0)),
            pl.BlockSpec((1, OUT), lambda b: (0, 0)),
            pl.BlockSpec((1, K), lambda b: (0, 0)),
        ],
        out_specs=pl.BlockSpec((BPG, K, OUT), lambda b: (b, 0, 0)),
        out_shape=jax.ShapeDtypeStruct((B, K, OUT), jnp.float32),
    )(x, adj, W1, a1s, a1d, W2, a2s, a2d, ones)
    return out


# speedup vs baseline: 2.6509x; 1.3062x over previous
import jax
import jax.numpy as jnp
from jax.experimental import pallas as pl

B, K, IN, H, OUT = 8, 256, 128, 64, 128
BPG = 4
ALPHA = 0.2
NEG_BIG = -9000000000000000.0


def _gat_block(h, mask, W_ref, a_ref, ones, nh):
    Wh = jax.lax.dot_general(h, W_ref[...], (((1,), (0,)), ((), ())),
                             preferred_element_type=jnp.float32)
    s = jax.lax.dot_general(Wh, a_ref[:, :nh], (((1,), (1,)), ((), ())),
                            preferred_element_type=jnp.float32)
    d = jax.lax.dot_general(a_ref[:, nh:], Wh, (((1,), (1,)), ((), ())),
                            preferred_element_type=jnp.float32)
    e = s + d
    e = jnp.maximum(e, ALPHA * e)
    att = jnp.where(mask, e, NEG_BIG)
    m = jnp.max(att, axis=1, keepdims=True)
    p = jnp.exp(att - m)
    rs = jax.lax.dot_general(p, ones, (((1,), (1,)), ((), ())),
                             preferred_element_type=jnp.float32)
    hp = jax.lax.dot_general(p, Wh, (((1,), (0,)), ((), ())),
                             preferred_element_type=jnp.float32)
    hp = hp * (1.0 / rs)
    return jnp.where(hp > 0, hp, jnp.exp(jnp.minimum(hp, 0.0)) - 1.0)


def _gat2_kernel(x_ref, adj_ref, W1_ref, a1_ref, W2_ref, a2_ref, out_ref):
    ones = jnp.ones((1, K), dtype=jnp.float32)
    for i in range(BPG):
        x = x_ref[i]
        mask = adj_ref[i] > 0
        h1 = _gat_block(x, mask, W1_ref, a1_ref, ones, H)
        out_ref[i] = _gat_block(h1, mask, W2_ref, a2_ref, ones, OUT)


def kernel(x, adj, W1, a1, W2, a2):
    grid = (B // BPG,)
    out = pl.pallas_call(
        _gat2_kernel,
        grid=grid,
        in_specs=[
            pl.BlockSpec((BPG, K, IN), lambda b: (b, 0, 0)),
            pl.BlockSpec((BPG, K, K), lambda b: (b, 0, 0)),
            pl.BlockSpec((IN, H), lambda b: (0, 0)),
            pl.BlockSpec((1, 2 * H), lambda b: (0, 0)),
            pl.BlockSpec((H, OUT), lambda b: (0, 0)),
            pl.BlockSpec((1, 2 * OUT), lambda b: (0, 0)),
        ],
        out_specs=pl.BlockSpec((BPG, K, OUT), lambda b: (b, 0, 0)),
        out_shape=jax.ShapeDtypeStruct((B, K, OUT), jnp.float32),
    )(x, adj, W1, a1.reshape(1, 2 * H), W2, a2.reshape(1, 2 * OUT))
    return out


# speedup vs baseline: 2.6805x; 1.0111x over previous
import jax
import jax.numpy as jnp
from jax.experimental import pallas as pl

B, K, IN, H, OUT = 8, 256, 128, 64, 128
BPG = 8
ALPHA = 0.2
NEG_BIG = -9000000000000000.0


def _gat_block(h, mask, W_ref, a_ref, ones, nh):
    Wh = jax.lax.dot_general(h, W_ref[...], (((1,), (0,)), ((), ())),
                             preferred_element_type=jnp.float32)
    s = jax.lax.dot_general(Wh, a_ref[:, :nh], (((1,), (1,)), ((), ())),
                            preferred_element_type=jnp.float32)
    d = jax.lax.dot_general(a_ref[:, nh:], Wh, (((1,), (1,)), ((), ())),
                            preferred_element_type=jnp.float32)
    e = s + d
    e = jnp.maximum(e, ALPHA * e)
    att = jnp.where(mask, e, NEG_BIG)
    m = jnp.max(att, axis=1, keepdims=True)
    p = jnp.exp(att - m)
    rs = jax.lax.dot_general(p, ones, (((1,), (1,)), ((), ())),
                             preferred_element_type=jnp.float32)
    hp = jax.lax.dot_general(p, Wh, (((1,), (0,)), ((), ())),
                             preferred_element_type=jnp.float32)
    hp = hp * (1.0 / rs)
    return jnp.where(hp > 0, hp, jnp.exp(jnp.minimum(hp, 0.0)) - 1.0)


def _gat2_kernel(x_ref, adj_ref, W1_ref, a1_ref, W2_ref, a2_ref, out_ref):
    ones = jnp.ones((1, K), dtype=jnp.float32)
    for i in range(BPG):
        x = x_ref[i]
        mask = adj_ref[i] > 0
        h1 = _gat_block(x, mask, W1_ref, a1_ref, ones, H)
        out_ref[i] = _gat_block(h1, mask, W2_ref, a2_ref, ones, OUT)


def kernel(x, adj, W1, a1, W2, a2):
    grid = (B // BPG,)
    out = pl.pallas_call(
        _gat2_kernel,
        grid=grid,
        in_specs=[
            pl.BlockSpec((BPG, K, IN), lambda b: (b, 0, 0)),
            pl.BlockSpec((BPG, K, K), lambda b: (b, 0, 0)),
            pl.BlockSpec((IN, H), lambda b: (0, 0)),
            pl.BlockSpec((1, 2 * H), lambda b: (0, 0)),
            pl.BlockSpec((H, OUT), lambda b: (0, 0)),
            pl.BlockSpec((1, 2 * OUT), lambda b: (0, 0)),
        ],
        out_specs=pl.BlockSpec((BPG, K, OUT), lambda b: (b, 0, 0)),
        out_shape=jax.ShapeDtypeStruct((B, K, OUT), jnp.float32),
    )(x, adj, W1, a1.reshape(1, 2 * H), W2, a2.reshape(1, 2 * OUT))
    return out
